# baseline (speedup 1.0000x reference)
import functools
import math
import jax
import jax.numpy as jnp
from jax import lax
from jax.experimental import pallas as pl
from jax.experimental.pallas import tpu as pltpu
from jax.experimental.pallas import tpu_sc as plsc

_M = 4
_G = 512
_NG = 32768
_GPT = 1024
_BLK = _GPT * _G
_CH = 64
_NCHUNK = _GPT // _CH
_CHW = _CH * _G
_NSET = _CH // 16
_INV_SCALE = 1.0 / math.sqrt(_G)
_NEG = -3.0e38

_mesh = plsc.VectorSubcoreMesh(
    core_axis_name="c", subcore_axis_name="s", num_cores=2, num_subcores=16
)


def _sc_body(x_hbm, q_hbm, buf_a, buf_b, ixv, si0, si1, so0, so1):
    w = lax.axis_index("s") * 2 + lax.axis_index("c")
    base = w * _BLK
    bufs = (buf_a, buf_b)
    sin = (si0, si1)
    sout = (so0, so1)
    lane = lax.iota(jnp.int32, 16)
    negv = jnp.full((16,), _NEG, jnp.float32)
    onesv = jnp.full((16,), 1.0, jnp.float32)
    zerov = jnp.zeros((16,), jnp.float32)

    def start_in(b, k):
        off = pl.multiple_of(base + k * _CHW, 256)
        return pltpu.async_copy(x_hbm.at[pl.ds(off, _CHW)], bufs[b], sin[b])

    def wait_in(b):
        pltpu.make_async_copy(x_hbm.at[pl.ds(0, _CHW)], bufs[b], sin[b]).wait()

    def start_out(b, k):
        off = pl.multiple_of(base + k * _CHW, 256)
        return pltpu.async_copy(bufs[b], q_hbm.at[pl.ds(off, _CHW)], sout[b])

    def wait_out(b):
        pltpu.make_async_copy(bufs[b], q_hbm.at[pl.ds(0, _CHW)], sout[b]).wait()

    set_bases = []
    for s in range(_NSET):
        u, hh = s // 2, s % 2
        set_bases.append(
            (u * 16384 + hh * 2 * 4096)
            + (lane // 8) * 4096
            + (lane % 8) * 128
        )

    def compute_chunk(b, k):
        buf = bufs[b]
        for s in range(_NSET):
            av = set_bases[s]
            init = []
            for j in range(4):
                init.extend((negv, av + j * 1024, av + j * 1024))

            def step(t, carry, buf=buf):
                out = []
                for j in range(4):
                    m, ri, addrv = carry[3 * j : 3 * j + 3]
                    for _ in range(8):
                        v = plsc.load_gather(buf, [addrv])
                        gt = v > m
                        m = jnp.where(gt, v, m)
                        ri = jnp.where(gt, addrv, ri)
                        addrv = addrv + 1
                    out.extend((m, ri, addrv))
                return tuple(out)

            res = lax.fori_loop(0, 16, step, tuple(init))
            m, ri = res[0], res[1]
            for j in range(1, 4):
                mj, rj = res[3 * j], res[3 * j + 1]
                gt = mj > m
                m = jnp.where(gt, mj, m)
                ri = jnp.where(gt, rj, ri)
            diff = ri - av
            g = ((diff >> 10) << 7) | (diff & 127)
            ixv[pl.ds(pl.multiple_of(k * _CH + s * 16, 16), 16)] = g

    handles = [start_in(0, 0), start_in(1, 1)]
    del handles

    def p1_pair(kk, c):
        for b in range(2):
            k = kk * 2 + b
            wait_in(b)
            compute_chunk(b, k)
            start_in(b, k + 2)
        return c

    lax.fori_loop(0, (_NCHUNK - 2) // 2, p1_pair, 0)
    for b in range(2):
        wait_in(b)
        compute_chunk(b, _NCHUNK - 2 + b)

    def zero_buf(buf):
        def zb(t, c, buf=buf):
            off = pl.multiple_of(t * 64, 16)
            for u in range(4):
                buf[pl.ds(off + u * 16, 16)] = zerov
            return c

        lax.fori_loop(0, _CHW // 64, zb, 0)

    zero_buf(buf_a)
    zero_buf(buf_b)

    def scat(b, k, val):
        buf = bufs[b]
        for s in range(_NSET):
            iv = ixv[pl.ds(pl.multiple_of(k * _CH + s * 16, 16), 16)]
            addr = set_bases[s] + ((iv >> 7) << 10) + (iv & 127)
            plsc.store_scatter(buf, [addr], val)

    for b in range(2):
        scat(b, b, onesv)
        start_out(b, b)

    def p2_pair(kk, c):
        for b in range(2):
            k = kk * 2 + 2 + b
            wait_out(b)
            scat(b, k - 2, zerov)
            scat(b, k, onesv)
            start_out(b, k)
        return c

    lax.fori_loop(0, (_NCHUNK - 2) // 2, p2_pair, 0)
    wait_out(0)
    wait_out(1)


_sc_quantize = functools.partial(
    pl.kernel,
    out_type=jax.ShapeDtypeStruct((_NG * _G,), jnp.float32),
    mesh=_mesh,
    compiler_params=pltpu.CompilerParams(needs_layout_passes=False),
    scratch_types=[
        pltpu.VMEM((_CHW,), jnp.float32),
        pltpu.VMEM((_CHW,), jnp.float32),
        pltpu.VMEM((_GPT,), jnp.int32),
        pltpu.SemaphoreType.DMA,
        pltpu.SemaphoreType.DMA,
        pltpu.SemaphoreType.DMA,
        pltpu.SemaphoreType.DMA,
    ],
)(_sc_body)


def _tc_body(x_ref, l_ref):
    l_ref[...] = (x_ref[...] * _INV_SCALE).reshape(l_ref.shape)


def _tc_scale(xT):
    n, h, w, c = xT.shape
    in_blk = (1, h, w, _G)
    out_blk = (1, 1, h, w, _G)
    return pl.pallas_call(
        _tc_body,
        grid=(n, _M),
        in_specs=[pl.BlockSpec(in_blk, lambda i, m: (i, 0, 0, m))],
        out_specs=pl.BlockSpec(out_blk, lambda i, m: (i, m, 0, 0, 0)),
        out_shape=jax.ShapeDtypeStruct((n, _M, h, w, _G), xT.dtype),
    )(xT)


def kernel(logit, temperature):
    n, c, h, w = logit.shape
    g = c // _M
    xT = logit.transpose(0, 2, 3, 1)
    x_phys = (
        xT.reshape(n, h, w // 8, 8, c // 128, 128)
        .transpose(0, 1, 2, 4, 3, 5)
        .reshape(-1)
    )
    q1 = _sc_quantize(x_phys)
    lT = _tc_scale(xT)
    qT = (
        q1.reshape(n, h, w // 8, c // 128, 8, 128)
        .transpose(0, 1, 2, 4, 3, 5)
        .reshape(n, h, w, c)
    )
    q = qT.transpose(0, 3, 1, 2)
    l = lT.transpose(0, 1, 4, 2, 3)
    return q, l


# speedup vs baseline: 1.0774x; 1.0774x over previous
import functools
import math
import jax
import jax.numpy as jnp
from jax import lax
from jax.experimental import pallas as pl
from jax.experimental.pallas import tpu as pltpu
from jax.experimental.pallas import tpu_sc as plsc

_M = 4
_G = 512
_NG = 32768
_GPT = 1024
_BLK = _GPT * _G
_CH = 64
_NCHUNK = _GPT // _CH
_CHW = _CH * _G
_NSET = _CH // 16
_INV_SCALE = 1.0 / math.sqrt(_G)
_NEG = -3.0e38

_mesh = plsc.VectorSubcoreMesh(
    core_axis_name="c", subcore_axis_name="s", num_cores=2, num_subcores=16
)


def _sc_body(x_hbm, q_hbm, buf_a, buf_b, qbuf_a, qbuf_b, ixv, si0, si1, sq0, sq1):
    w = lax.axis_index("s") * 2 + lax.axis_index("c")
    base = w * _BLK
    bufs = (buf_a, buf_b)
    qbufs = (qbuf_a, qbuf_b)
    sin = (si0, si1)
    sqs = (sq0, sq1)
    lane = lax.iota(jnp.int32, 16)
    negv = jnp.full((16,), _NEG, jnp.float32)
    onesv = jnp.full((16,), 1.0, jnp.float32)
    zerov = jnp.zeros((16,), jnp.float32)

    def start_in(b, k):
        off = pl.multiple_of(base + k * _CHW, 256)
        return pltpu.async_copy(x_hbm.at[pl.ds(off, _CHW)], bufs[b], sin[b])

    def wait_in(b):
        pltpu.make_async_copy(x_hbm.at[pl.ds(0, _CHW)], bufs[b], sin[b]).wait()

    def start_qout(u, k):
        off = pl.multiple_of(base + k * _CHW + u * 16384, 256)
        return pltpu.async_copy(qbufs[u], q_hbm.at[pl.ds(off, 16384)], sqs[u])

    def wait_qout(u):
        pltpu.make_async_copy(qbufs[u], q_hbm.at[pl.ds(0, 16384)], sqs[u]).wait()

    set_bases = []
    for s in range(_NSET):
        u, hh = s // 2, s % 2
        set_bases.append(
            (u * 16384 + hh * 2 * 4096)
            + (lane // 8) * 4096
            + (lane % 8) * 128
        )

    def argmax_set(b, s):
        buf = bufs[b]
        av = set_bases[s]
        init = []
        for j in range(4):
            init.extend((negv, av + j * 1024, av + j * 1024))

        def step(t, carry, buf=buf):
            out = []
            for j in range(4):
                m, ri, addrv = carry[3 * j : 3 * j + 3]
                for _ in range(8):
                    v = plsc.load_gather(buf, [addrv])
                    gt = v > m
                    m = jnp.where(gt, v, m)
                    ri = jnp.where(gt, addrv, ri)
                    addrv = addrv + 1
                out.extend((m, ri, addrv))
            return tuple(out)

        res = lax.fori_loop(0, 16, step, tuple(init))
        m, ri = res[0], res[1]
        for j in range(1, 4):
            mj, rj = res[3 * j], res[3 * j + 1]
            gt = mj > m
            m = jnp.where(gt, mj, m)
            ri = jnp.where(gt, rj, ri)
        diff = ri - av
        return ((diff >> 10) << 7) | (diff & 127)

    def scat_set(u, s, iv, val):
        addr = (set_bases[s] - u * 16384) + ((iv >> 7) << 10) + (iv & 127)
        plsc.store_scatter(qbufs[u], [addr], val)

    def zero_qbuf(u):
        def zb(t, c, u=u):
            off = pl.multiple_of(t * 64, 16)
            for j in range(4):
                qbufs[u][pl.ds(off + j * 16, 16)] = zerov
            return c

        lax.fori_loop(0, 16384 // 64, zb, 0)

    def chunk_body(b, k, qwait):
        wait_in(b)
        for u in range(2):
            if qwait:
                wait_qout(u)
                for sl in range(2):
                    old = ixv[pl.ds(u * 32 + sl * 16, 16)]
                    scat_set(u, u * 2 + sl, old, zerov)
            for sl in range(2):
                s = u * 2 + sl
                g = argmax_set(b, s)
                ixv[pl.ds(u * 32 + sl * 16, 16)] = g
                scat_set(u, s, g, onesv)
            start_qout(u, k)

    start_in(0, 0)
    start_in(1, 1)
    zero_qbuf(0)
    zero_qbuf(1)

    chunk_body(0, 0, qwait=False)
    start_in(0, 2)
    chunk_body(1, 1, qwait=True)
    start_in(1, 3)

    def pair(kk, c):
        for b in range(2):
            k = kk * 2 + 2 + b
            chunk_body(b, k, qwait=True)
            start_in(b, k + 2)
        return c

    lax.fori_loop(0, (_NCHUNK - 4) // 2, pair, 0)
    for b in range(2):
        chunk_body(b, _NCHUNK - 2 + b, qwait=True)
    wait_qout(0)
    wait_qout(1)


_sc_quantize = functools.partial(
    pl.kernel,
    out_type=jax.ShapeDtypeStruct((_NG * _G,), jnp.float32),
    mesh=_mesh,
    compiler_params=pltpu.CompilerParams(needs_layout_passes=False),
    scratch_types=[
        pltpu.VMEM((_CHW,), jnp.float32),
        pltpu.VMEM((_CHW,), jnp.float32),
        pltpu.VMEM((16384,), jnp.float32),
        pltpu.VMEM((16384,), jnp.float32),
        pltpu.VMEM((64,), jnp.int32),
        pltpu.SemaphoreType.DMA,
        pltpu.SemaphoreType.DMA,
        pltpu.SemaphoreType.DMA,
        pltpu.SemaphoreType.DMA,
    ],
)(_sc_body)


def _tc_body(x_ref, l_ref):
    l_ref[...] = (x_ref[...] * _INV_SCALE).reshape(l_ref.shape)


def _tc_scale(xT):
    n, h, w, c = xT.shape
    in_blk = (1, h, w, _G)
    out_blk = (1, 1, h, w, _G)
    return pl.pallas_call(
        _tc_body,
        grid=(n, _M),
        in_specs=[pl.BlockSpec(in_blk, lambda i, m: (i, 0, 0, m))],
        out_specs=pl.BlockSpec(out_blk, lambda i, m: (i, m, 0, 0, 0)),
        out_shape=jax.ShapeDtypeStruct((n, _M, h, w, _G), xT.dtype),
    )(xT)


def kernel(logit, temperature):
    n, c, h, w = logit.shape
    g = c // _M
    xT = logit.transpose(0, 2, 3, 1)
    x_phys = (
        xT.reshape(n, h, w // 8, 8, c // 128, 128)
        .transpose(0, 1, 2, 4, 3, 5)
        .reshape(-1)
    )
    q1 = _sc_quantize(x_phys)
    lT = _tc_scale(xT)
    qT = (
        q1.reshape(n, h, w // 8, c // 128, 8, 128)
        .transpose(0, 1, 2, 4, 3, 5)
        .reshape(n, h, w, c)
    )
    q = qT.transpose(0, 3, 1, 2)
    l = lT.transpose(0, 1, 4, 2, 3)
    return q, l
